# Initial kernel scaffold; baseline (speedup 1.0000x reference)
import jax
import jax.numpy as jnp
from jax.experimental import pallas as pl


def kernel(x, edge_index, W1, b1, W2, b2, W3, b3, Wih0, Whh0, bih0, bhh0, Wih1, Whh1, bih1, bhh1, Wr, br, We, be):
    raise NotImplementedError("write your pallas kernel here")


# speedup vs baseline: 9.2637x; 9.2637x over previous
import functools

import jax
import jax.numpy as jnp
from jax import lax
from jax.experimental import pallas as pl
from jax.experimental.pallas import tpu as pltpu
from jax.experimental.pallas import tpu_sc as plsc

N = 10000
E = 320000
F = 128
HID = 128
CT = 16
NN = 2000
T = 5
LH = 256
RS = 256
NT = 4

NUM_CORES = 2
NUM_SUBCORES = 16
NUM_TILES = NUM_CORES * NUM_SUBCORES
EDGES_PER_TILE = E // NUM_TILES
CHUNK = 80
NCHUNK = EDGES_PER_TILE // CHUNK
ROWS_PER_TILE = N // NUM_SUBCORES


def _make_sc_aggregate(D):
    mesh = plsc.VectorSubcoreMesh(core_axis_name="c", subcore_axis_name="s")

    @functools.partial(
        pl.kernel,
        mesh=mesh,
        out_type=jax.ShapeDtypeStruct(
            (NUM_CORES, NUM_SUBCORES, ROWS_PER_TILE, D), jnp.float32),
        scratch_types=[
            pltpu.VMEM((CHUNK,), jnp.int32),
            pltpu.VMEM((CHUNK,), jnp.int32),
            pltpu.VMEM((CHUNK, D), jnp.float32),
            pltpu.VMEM_SHARED((N, D), jnp.float32),
            pltpu.SemaphoreType.DMA,
        ],
    )
    def agg(table_hbm, src_hbm, dst_hbm, zeros_hbm, out_hbm,
            sidx, didx, rows, acc, sem):
        c = lax.axis_index("c")
        s = lax.axis_index("s")
        wid = c * NUM_SUBCORES + s
        pltpu.sync_copy(zeros_hbm, acc.at[pl.ds(s * ROWS_PER_TILE, ROWS_PER_TILE)])
        plsc.subcore_barrier()
        ebase = wid * EDGES_PER_TILE

        def body(j, carry):
            base = ebase + j * CHUNK
            pltpu.sync_copy(src_hbm.at[pl.ds(base, CHUNK)], sidx)
            pltpu.sync_copy(dst_hbm.at[pl.ds(base, CHUNK)], didx)
            pltpu.async_copy(table_hbm.at[sidx], rows, sem).wait()
            pltpu.sync_copy(rows, acc.at[didx], add=True)
            return carry

        lax.fori_loop(0, NCHUNK, body, 0)
        plsc.subcore_barrier()
        pltpu.sync_copy(acc.at[pl.ds(s * ROWS_PER_TILE, ROWS_PER_TILE)],
                        out_hbm.at[c].at[s])

    return agg


_sc_agg128 = _make_sc_aggregate(HID)


def _make_sc_degree():
    mesh = plsc.VectorSubcoreMesh(core_axis_name="c", subcore_axis_name="s")

    @functools.partial(
        pl.kernel,
        mesh=mesh,
        out_type=jax.ShapeDtypeStruct(
            (NUM_CORES, NUM_SUBCORES, ROWS_PER_TILE, HID), jnp.float32),
        scratch_types=[
            pltpu.VMEM((CHUNK,), jnp.int32),
            pltpu.VMEM((CHUNK, HID), jnp.float32),
            pltpu.VMEM_SHARED((N, HID), jnp.float32),
        ],
    )
    def degk(dst_hbm, ones_hbm, zeros_hbm, out_hbm, didx, rows, acc):
        c = lax.axis_index("c")
        s = lax.axis_index("s")
        wid = c * NUM_SUBCORES + s
        pltpu.sync_copy(ones_hbm, rows)
        pltpu.sync_copy(zeros_hbm, acc.at[pl.ds(s * ROWS_PER_TILE, ROWS_PER_TILE)])
        plsc.subcore_barrier()
        ebase = wid * EDGES_PER_TILE

        def body(j, carry):
            pltpu.sync_copy(dst_hbm.at[pl.ds(ebase + j * CHUNK, CHUNK)], didx)
            pltpu.sync_copy(rows, acc.at[didx], add=True)
            return carry

        lax.fori_loop(0, NCHUNK, body, 0)
        plsc.subcore_barrier()
        pltpu.sync_copy(acc.at[pl.ds(s * ROWS_PER_TILE, ROWS_PER_TILE)],
                        out_hbm.at[c].at[s])

    return degk


_sc_degree = _make_sc_degree()


_RB = 2000
_GRID = N // _RB

_lrelu = lambda v: jnp.where(v >= 0, v, 0.01 * v)


def _full(shape):
    return pl.BlockSpec(shape, lambda i: tuple(0 for _ in shape))


def _rows2(d):
    return pl.BlockSpec((_RB, d), lambda i: (i, 0))


def _k1_body(x_ref, w_ref, dp_ref, hs_ref, dis_ref):
    deg = dp_ref[0, :, 0:1] + dp_ref[1, :, 0:1] + 1.0
    dis = lax.rsqrt(deg)
    h = jnp.dot(x_ref[...], w_ref[...], preferred_element_type=jnp.float32)
    hs_ref[...] = dis * h
    dis_ref[...] = dis


def _k1(x, W1, degp):
    return pl.pallas_call(
        _k1_body,
        grid=(_GRID,),
        in_specs=[_rows2(F), _full((F, HID)),
                  pl.BlockSpec((NUM_CORES, _RB, HID), lambda i: (0, i, 0))],
        out_specs=[_rows2(HID), _rows2(1)],
        out_shape=[jax.ShapeDtypeStruct((N, HID), jnp.float32),
                   jax.ShapeDtypeStruct((N, 1), jnp.float32)],
    )(x, W1, degp)


def _k2_body(hs_ref, ap_ref, dis_ref, b_ref, w_ref, h1_ref, hs2_ref):
    dis = dis_ref[...]
    aggd = ap_ref[0] + ap_ref[1] + hs_ref[...]
    h1 = _lrelu(dis * aggd + b_ref[...])
    h1_ref[...] = h1
    h2p = jnp.dot(h1, w_ref[...], preferred_element_type=jnp.float32)
    hs2_ref[...] = dis * h2p


def _k2(hs1, agg1, dis, b1, W2):
    return pl.pallas_call(
        _k2_body,
        grid=(_GRID,),
        in_specs=[_rows2(HID),
                  pl.BlockSpec((NUM_CORES, _RB, HID), lambda i: (0, i, 0)),
                  _rows2(1), _full((1, HID)), _full((HID, HID))],
        out_specs=[_rows2(HID), _rows2(HID)],
        out_shape=[jax.ShapeDtypeStruct((N, HID), jnp.float32),
                   jax.ShapeDtypeStruct((N, HID), jnp.float32)],
    )(hs1, agg1, dis, b1, W2)


def _k3_body(hs_ref, ap_ref, dis_ref, b_ref, h1_ref, u2_ref):
    dis = dis_ref[...]
    h_ = dis * (ap_ref[0] + ap_ref[1] + hs_ref[...]) + b_ref[...]
    h2 = (_lrelu(h_) + h1_ref[...]) * 0.5
    u2_ref[...] = dis * h2


def _k3(hs2, agg2, dis, b2, h1):
    return pl.pallas_call(
        _k3_body,
        grid=(_GRID,),
        in_specs=[_rows2(HID),
                  pl.BlockSpec((NUM_CORES, _RB, HID), lambda i: (0, i, 0)),
                  _rows2(1), _full((1, HID)), _rows2(HID)],
        out_specs=_rows2(HID),
        out_shape=jax.ShapeDtypeStruct((N, HID), jnp.float32),
    )(hs2, agg2, dis, b2, h1)


def _k4_body(u2_ref, ap_ref, dis_ref, w_ref, b_ref, h3_ref):
    dis = dis_ref[...]
    t = ap_ref[0] + ap_ref[1] + u2_ref[...]
    h3p = jnp.dot(t, w_ref[...], preferred_element_type=jnp.float32)
    h3_ref[...] = _lrelu(dis * h3p + b_ref[...])


def _k4(u2, agg3, dis, W3, b3):
    return pl.pallas_call(
        _k4_body,
        grid=(_GRID,),
        in_specs=[_rows2(HID),
                  pl.BlockSpec((NUM_CORES, _RB, HID), lambda i: (0, i, 0)),
                  _rows2(1), _full((HID, CT)), _full((1, CT))],
        out_specs=_rows2(CT),
        out_shape=jax.ShapeDtypeStruct((N, CT), jnp.float32),
    )(u2, agg3, dis, W3, b3)


_KC = 1280
_KGRID = (NN * CT) // _KC


def _k5_body(seq_ref, w_ref, bih_ref, bhh_ref, out_ref):
    @pl.when(pl.program_id(0) == 0)
    def _():
        out_ref[...] = jnp.broadcast_to(bih_ref[...] + bhh_ref[...], out_ref.shape)

    out_ref[...] += jnp.dot(seq_ref[...], w_ref[...],
                            preferred_element_type=jnp.float32)


def _k5(seq, Wih0T, bih0, bhh0):
    return pl.pallas_call(
        _k5_body,
        grid=(_KGRID,),
        in_specs=[pl.BlockSpec((T, _KC), lambda k: (0, k)),
                  pl.BlockSpec((_KC, 4 * LH), lambda k: (k, 0)),
                  _full((1, 4 * LH)), _full((1, 4 * LH))],
        out_specs=pl.BlockSpec((T, 4 * LH), lambda k: (0, 0)),
        out_shape=jax.ShapeDtypeStruct((T, 4 * LH), jnp.float32),
    )(seq, Wih0T, bih0, bhh0)


def _lstm_steps(gx, WhhT_ref):
    h = jnp.zeros((1, LH), jnp.float32)
    c = jnp.zeros((1, LH), jnp.float32)
    hs = []
    for t in range(T):
        g = gx[t] + jnp.dot(h, WhhT_ref[...], preferred_element_type=jnp.float32)
        i = jax.nn.sigmoid(g[:, 0 * LH:1 * LH])
        f = jax.nn.sigmoid(g[:, 1 * LH:2 * LH])
        gg = jnp.tanh(g[:, 2 * LH:3 * LH])
        o = jax.nn.sigmoid(g[:, 3 * LH:4 * LH])
        c = f * c + i * gg
        h = o * jnp.tanh(c)
        hs.append(h)
    return hs


def _k6_body(gx0_ref, whh0_ref, wih1_ref, bih1_ref, bhh1_ref, whh1_ref,
             wr_ref, br_ref, we_ref, be_ref, out_ref):
    gx0 = [gx0_ref[t:t + 1, :] for t in range(T)]
    hs0 = _lstm_steps(gx0, whh0_ref)
    b1 = bih1_ref[...] + bhh1_ref[...]
    gx1 = [jnp.dot(hs0[t], wih1_ref[...], preferred_element_type=jnp.float32) + b1
           for t in range(T)]
    hs1 = _lstm_steps(gx1, whh1_ref)
    v = _lrelu(hs1[-1])
    v = _lrelu(jnp.dot(v, wr_ref[...], preferred_element_type=jnp.float32)
               + br_ref[...])
    out_ref[...] = (jnp.dot(v, we_ref[...], preferred_element_type=jnp.float32)
                    + be_ref[...])


def _k6(gx0, Whh0T, Wih1T, bih1, bhh1, Whh1T, WrT, br, WeT, be):
    return pl.pallas_call(
        _k6_body,
        grid=(1,),
        in_specs=[_full((T, 4 * LH)), _full((LH, 4 * LH)), _full((LH, 4 * LH)),
                  _full((1, 4 * LH)), _full((1, 4 * LH)), _full((LH, 4 * LH)),
                  _full((LH, RS)), _full((1, RS)), _full((RS, NT)),
                  _full((1, NT))],
        out_specs=_full((1, NT)),
        out_shape=jax.ShapeDtypeStruct((1, NT), jnp.float32),
    )(gx0, Whh0T, Wih1T, bih1, bhh1, Whh1T, WrT, br, WeT, be)


def kernel(x, edge_index, W1, b1, W2, b2, W3, b3, Wih0, Whh0, bih0, bhh0,
           Wih1, Whh1, bih1, bhh1, Wr, br, We, be):
    src = edge_index[0]
    dst = edge_index[1]
    z128 = jnp.zeros((ROWS_PER_TILE, HID), jnp.float32)
    o128 = jnp.ones((CHUNK, HID), jnp.float32)

    degp = _sc_degree(dst, o128, z128).reshape(NUM_CORES, N, HID)
    hs1, dis = _k1(x, W1, degp)
    agg1 = _sc_agg128(hs1, src, dst, z128).reshape(NUM_CORES, N, HID)
    h1, hs2 = _k2(hs1, agg1, dis, b1.reshape(1, HID), W2)
    agg2 = _sc_agg128(hs2, src, dst, z128).reshape(NUM_CORES, N, HID)
    u2 = _k3(hs2, agg2, dis, b2.reshape(1, HID), h1)
    agg3 = _sc_agg128(u2, src, dst, z128).reshape(NUM_CORES, N, HID)
    h3 = _k4(u2, agg3, dis, W3, b3.reshape(1, CT))
    seq = h3.reshape(T, NN * CT)
    gx0 = _k5(seq, Wih0.T, bih0.reshape(1, -1), bhh0.reshape(1, -1))
    out = _k6(gx0, Whh0.T, Wih1.T, bih1.reshape(1, -1), bhh1.reshape(1, -1),
              Whh1.T, Wr.T, br.reshape(1, -1), We.T, be.reshape(1, -1))
    return out.reshape(NT)
